# Initial kernel scaffold; baseline (speedup 1.0000x reference)
import jax
import jax.numpy as jnp
from jax.experimental import pallas as pl


def kernel(X, gate_weights, output_weights, output_scale):
    raise NotImplementedError("write your pallas kernel here")


# speedup vs baseline: 355.8180x; 355.8180x over previous
import jax
import jax.numpy as jnp
from jax.experimental import pallas as pl
from jax.experimental.pallas import tpu as pltpu

N_FEAT = 128
N_GATES = 64
MAX_CONN = N_FEAT + 2 + N_GATES
SUB = 8
LANES = 256
BLK = SUB * LANES


def _topk_tc_kernel(gw_ref, idx_ref):
    gw = gw_ref[...]
    col = jax.lax.broadcasted_iota(jnp.int32, gw.shape, 1)
    row = jax.lax.broadcasted_iota(jnp.int32, gw.shape, 0)
    valid = col < (row + N_FEAT + 2)
    logits = jnp.where(valid, gw, -1e30)
    m = jnp.max(logits, axis=1, keepdims=True)
    e = jnp.exp(logits - m)
    probs = e / jnp.sum(e, axis=1, keepdims=True)
    probs = jnp.where(valid, probs, -1.0)
    big = jnp.int32(1 << 30)
    m1 = jnp.max(probs, axis=1, keepdims=True)
    i1 = jnp.min(jnp.where(probs == m1, col, big), axis=1, keepdims=True)
    probs2 = jnp.where(col == i1, -1.0, probs)
    m2 = jnp.max(probs2, axis=1, keepdims=True)
    i2 = jnp.min(jnp.where(probs2 == m2, col, big), axis=1, keepdims=True)
    idx_ref[...] = jnp.concatenate([i1, i2], axis=1)


def _chain_kernel(idx_ref, x_ref, g_ref, avail_ref):
    avail_ref[0:N_FEAT] = x_ref[...]
    avail_ref[N_FEAT] = jnp.zeros((SUB, LANES), jnp.float32)
    avail_ref[N_FEAT + 1] = jnp.ones((SUB, LANES), jnp.float32)

    def step(g, carry):
        ia = idx_ref[g, 0]
        ib = idx_ref[g, 1]
        a = avail_ref[ia]
        b = avail_ref[ib]
        avail_ref[N_FEAT + 2 + g] = 1.0 - a * b
        return carry

    jax.lax.fori_loop(0, N_GATES, step, 0)
    g_ref[...] = avail_ref[N_FEAT + 2:]


def _matmul_kernel(wt_ref, g_ref, scale_ref, out_ref):
    out_ref[...] = (
        jnp.dot(wt_ref[...], g_ref[...], preferred_element_type=jnp.float32)
        * scale_ref[...]
    )


def kernel(X, gate_weights, output_weights, output_scale):
    n = X.shape[0]
    n_out = output_weights.shape[1]
    nblk = n // BLK

    idx = pl.pallas_call(
        _topk_tc_kernel,
        out_shape=jax.ShapeDtypeStruct((N_GATES, 2), jnp.int32),
    )(gate_weights)

    xt = X.T.reshape(N_FEAT, n // LANES, LANES)
    g3 = pl.pallas_call(
        _chain_kernel,
        grid=(nblk,),
        in_specs=[
            pl.BlockSpec(memory_space=pltpu.SMEM),
            pl.BlockSpec((N_FEAT, SUB, LANES), lambda i: (0, i, 0)),
        ],
        out_specs=pl.BlockSpec((N_GATES, SUB, LANES), lambda i: (0, i, 0)),
        out_shape=jax.ShapeDtypeStruct((N_GATES, n // LANES, LANES), jnp.float32),
        scratch_shapes=[pltpu.VMEM((MAX_CONN, SUB, LANES), jnp.float32)],
    )(idx, xt)
    g2 = g3.reshape(N_GATES, n)

    outt = pl.pallas_call(
        _matmul_kernel,
        grid=(nblk,),
        in_specs=[
            pl.BlockSpec((n_out, N_GATES), lambda i: (0, 0)),
            pl.BlockSpec((N_GATES, BLK), lambda i: (0, i)),
            pl.BlockSpec((n_out, 1), lambda i: (0, 0)),
        ],
        out_specs=pl.BlockSpec((n_out, BLK), lambda i: (0, i)),
        out_shape=jax.ShapeDtypeStruct((n_out, n), jnp.float32),
    )(output_weights.T, g2, output_scale.reshape(n_out, 1))
    return outt.T
